# baseline (speedup 1.0000x reference)
import functools

import jax
import jax.numpy as jnp
from jax.experimental import pallas as pl
from jax.experimental.pallas import tpu as pltpu


def _argmax_matmul_kernel(pos_ref, x_ref, emb_ref, out_ref, max_sc, idx_sc,
                          *, vt: int, nt: int, v: int):
    i = pl.program_id(0)

    @pl.when(i == 0)
    def _init():
        max_sc[...] = jnp.full_like(max_sc[...], -jnp.inf)
        idx_sc[...] = jnp.zeros_like(idx_sc[...])

    xs = x_ref[0]
    logits = jax.lax.dot_general(
        xs, emb_ref[...], (((1,), (1,)), ((), ())),
        preferred_element_type=jnp.float32)

    col = i * vt + jax.lax.broadcasted_iota(jnp.int32, logits.shape, 1)
    logits = jnp.where(col < v, logits, -jnp.inf)

    local_max = jnp.max(logits, axis=1, keepdims=True)
    local_idx = jnp.argmax(logits, axis=1).astype(jnp.int32)[:, None] + i * vt

    better = local_max > max_sc[...]
    idx_sc[...] = jnp.where(better, local_idx, idx_sc[...])
    max_sc[...] = jnp.where(better, local_max, max_sc[...])

    @pl.when(i == nt - 1)
    def _done():
        out_ref[...] = idx_sc[...]


def kernel(embedding, x, output_pos, temperature, topp, topk, embedding_bias=None):
    v, d = embedding.shape
    b, s, _ = x.shape
    vt = 7168
    nt = pl.cdiv(v, vt)

    xt = jnp.swapaxes(x, 0, 1)
    pos = output_pos.astype(jnp.int32)

    grid_spec = pltpu.PrefetchScalarGridSpec(
        num_scalar_prefetch=1,
        grid=(nt,),
        in_specs=[
            pl.BlockSpec((1, b, d), lambda i, pos_ref: (pos_ref[0], 0, 0)),
            pl.BlockSpec((vt, d), lambda i, pos_ref: (i, 0)),
        ],
        out_specs=pl.BlockSpec((b, 1), lambda i, pos_ref: (0, 0)),
        scratch_shapes=[
            pltpu.VMEM((b, 1), jnp.float32),
            pltpu.VMEM((b, 1), jnp.int32),
        ],
    )
    out = pl.pallas_call(
        functools.partial(_argmax_matmul_kernel, vt=vt, nt=nt, v=v),
        grid_spec=grid_spec,
        out_shape=jax.ShapeDtypeStruct((b, 1), jnp.int32),
        compiler_params=pltpu.CompilerParams(
            vmem_limit_bytes=100 * 1024 * 1024),
    )(pos, xt, embedding)
    return out[:, 0]


# speedup vs baseline: 1.0090x; 1.0090x over previous
import functools

import jax
import jax.numpy as jnp
from jax.experimental import pallas as pl
from jax.experimental.pallas import tpu as pltpu


def _argmax_matmul_kernel(pos_ref, x_ref, emb_ref, out_ref, max_sc, idx_sc,
                          *, vt: int, nt: int, v: int):
    i = pl.program_id(0)

    @pl.when(i == 0)
    def _init():
        max_sc[...] = jnp.full_like(max_sc[...], -jnp.inf)
        idx_sc[...] = jnp.zeros_like(idx_sc[...])

    xs = x_ref[...]
    logits = jax.lax.dot_general(
        xs, emb_ref[...], (((1,), (1,)), ((), ())),
        preferred_element_type=jnp.float32)

    col = i * vt + jax.lax.broadcasted_iota(jnp.int32, logits.shape, 1)
    logits = jnp.where(col < v, logits, -jnp.inf)

    local_max = jnp.max(logits, axis=1, keepdims=True)
    local_idx = jnp.argmax(logits, axis=1).astype(jnp.int32)[:, None] + i * vt

    better = local_max > max_sc[...]
    idx_sc[...] = jnp.where(better, local_idx, idx_sc[...])
    max_sc[...] = jnp.where(better, local_max, max_sc[...])

    @pl.when(i == nt - 1)
    def _done():
        out_ref[...] = idx_sc[...]


def kernel(embedding, x, output_pos, temperature, topp, topk, embedding_bias=None):
    v, d = embedding.shape
    b, s, _ = x.shape
    vt = 4096
    nt = pl.cdiv(v, vt)

    xt = x.reshape(b, s * d)
    pos = output_pos.astype(jnp.int32)

    grid_spec = pltpu.PrefetchScalarGridSpec(
        num_scalar_prefetch=1,
        grid=(nt,),
        in_specs=[
            pl.BlockSpec((b, d), lambda i, pos_ref: (0, pos_ref[0])),
            pl.BlockSpec((vt, d), lambda i, pos_ref: (i, 0)),
        ],
        out_specs=pl.BlockSpec((b, 1), lambda i, pos_ref: (0, 0)),
        scratch_shapes=[
            pltpu.VMEM((b, 1), jnp.float32),
            pltpu.VMEM((b, 1), jnp.int32),
        ],
    )
    out = pl.pallas_call(
        functools.partial(_argmax_matmul_kernel, vt=vt, nt=nt, v=v),
        grid_spec=grid_spec,
        out_shape=jax.ShapeDtypeStruct((b, 1), jnp.int32),
        compiler_params=pltpu.CompilerParams(
            vmem_limit_bytes=100 * 1024 * 1024),
    )(pos, xt, embedding)
    return out[:, 0]
